# baseline (speedup 1.0000x reference)
import functools

import jax
import jax.numpy as jnp
from jax import lax
from jax.experimental import pallas as pl
from jax.experimental.pallas import tpu as pltpu

B, R, KT, C, H = 4096, 8, 2, 8, 128
K = KT + 1
BB = 256


def _fused_body(targets_ref,
                o_pos_ref, t_pos_ref, c_pos_ref,
                o_typ_ref, t_typ_ref, c_typ_ref,
                o_siz_ref, t_siz_ref, c_siz_ref,
                o_col_ref, t_col_ref, c_col_ref,
                out_pos_ref, out_typ_ref, out_siz_ref, out_col_ref):
    t = targets_ref[0, 0, :]
    onehot = (t[:, None] ==
              lax.broadcasted_iota(jnp.int32, (BB, C), 1)).astype(jnp.float32)

    for o_ref, te_ref, ca_ref, out_ref in (
            (o_pos_ref, t_pos_ref, c_pos_ref, out_pos_ref),
            (o_typ_ref, t_typ_ref, c_typ_ref, out_typ_ref),
            (o_siz_ref, t_siz_ref, c_siz_ref, out_siz_ref),
            (o_col_ref, t_col_ref, c_col_ref, out_col_ref)):
        o = o_ref[...]
        te = te_ref[...]
        ca = ca_ref[...]
        chosen = jnp.sum(onehot[:, :, None] * ca, axis=1)
        tf = jnp.concatenate([te, chosen[:, None, :]], axis=1)
        diff = o - tf[:, None, :, :]
        scores = -jnp.mean(diff * diff, axis=(2, 3))
        m = jnp.max(scores, axis=-1, keepdims=True)
        e = jnp.exp(scores - m)
        w = e / jnp.sum(e, axis=-1, keepdims=True)
        out_ref[...] = jnp.sum(o * w[:, :, None, None], axis=1)


def kernel(outputs_position, tests_position, candidates_position,
           outputs_type, tests_type, candidates_type,
           outputs_size, tests_size, candidates_size,
           outputs_color, tests_color, candidates_color,
           targets):
    nb = B // BB
    tgt = targets.astype(jnp.int32).reshape(nb, 1, BB)

    o_spec = pl.BlockSpec((BB, R, K, H), lambda i: (i, 0, 0, 0))
    t_spec = pl.BlockSpec((BB, KT, H), lambda i: (i, 0, 0))
    c_spec = pl.BlockSpec((BB, C, H), lambda i: (i, 0, 0))
    tgt_spec = pl.BlockSpec((1, 1, BB), lambda i: (i, 0, 0))
    out_spec = pl.BlockSpec((BB, K, H), lambda i: (i, 0, 0))

    out_shape = jax.ShapeDtypeStruct((B, K, H), jnp.float32)
    grid_spec = pl.GridSpec(
        grid=(nb,),
        in_specs=[tgt_spec] + [o_spec, t_spec, c_spec] * 4,
        out_specs=[out_spec] * 4,
    )
    outs = pl.pallas_call(
        _fused_body,
        grid_spec=grid_spec,
        out_shape=[out_shape] * 4,
        compiler_params=pltpu.CompilerParams(
            dimension_semantics=("arbitrary",)),
    )(tgt,
      outputs_position, tests_position, candidates_position,
      outputs_type, tests_type, candidates_type,
      outputs_size, tests_size, candidates_size,
      outputs_color, tests_color, candidates_color)
    return tuple(outs)


# speedup vs baseline: 1.3307x; 1.3307x over previous
import jax
import jax.numpy as jnp
from jax import lax
from jax.experimental import pallas as pl
from jax.experimental.pallas import tpu as pltpu

B, R, KT, C, H = 4096, 8, 2, 8, 128
K = KT + 1
BB = 128


def _fused_body(targets_ref, *refs):
    outs = refs[12:]
    t = targets_ref[0]
    tb = jnp.broadcast_to(t[:, :, None], (BB, C, H))
    cidx = lax.broadcasted_iota(jnp.int32, (BB, C, H), 1)
    mask = (tb == cidx).astype(jnp.float32)

    ones_h = jnp.full((H, H), -1.0 / (K * H), dtype=jnp.float32)

    for a in range(4):
        o_ref, te_ref, ca = refs[3 * a:3 * a + 3]
        o = o_ref[...]
        chosen = jnp.sum(mask * ca[...], axis=1, keepdims=True)
        tf = jnp.concatenate([te_ref[...], chosen], axis=1)

        d = o - tf[:, None, :, :]
        acc = jnp.sum(d * d, axis=2)

        s = jnp.dot(acc.reshape(BB * R, H), ones_h,
                    preferred_element_type=jnp.float32).reshape(BB, R, H)
        m = jnp.max(s, axis=1, keepdims=True)
        e = jnp.exp(s - m)
        w = e / jnp.sum(e, axis=1, keepdims=True)

        outs[a][...] = jnp.sum(o * w[:, :, None, :], axis=1)


def kernel(outputs_position, tests_position, candidates_position,
           outputs_type, tests_type, candidates_type,
           outputs_size, tests_size, candidates_size,
           outputs_color, tests_color, candidates_color,
           targets):
    nb = B // BB
    tgt = targets.astype(jnp.int32).reshape(nb, BB, 1)

    o_spec = pl.BlockSpec((BB, R, K, H), lambda i: (i, 0, 0, 0))
    t_spec = pl.BlockSpec((BB, KT, H), lambda i: (i, 0, 0))
    c_spec = pl.BlockSpec((BB, C, H), lambda i: (i, 0, 0))
    tgt_spec = pl.BlockSpec((1, BB, 1), lambda i: (i, 0, 0))
    out_spec = pl.BlockSpec((BB, K, H), lambda i: (i, 0, 0))

    in_specs = [tgt_spec]
    operands = [tgt]
    for o, te, ca in ((outputs_position, tests_position, candidates_position),
                      (outputs_type, tests_type, candidates_type),
                      (outputs_size, tests_size, candidates_size),
                      (outputs_color, tests_color, candidates_color)):
        in_specs += [o_spec, t_spec, c_spec]
        operands += [o, te, ca]

    out_shape = jax.ShapeDtypeStruct((B, K, H), jnp.float32)
    grid_spec = pl.GridSpec(
        grid=(nb,),
        in_specs=in_specs,
        out_specs=[out_spec] * 4,
    )
    outs = pl.pallas_call(
        _fused_body,
        grid_spec=grid_spec,
        out_shape=[out_shape] * 4,
        compiler_params=pltpu.CompilerParams(
            dimension_semantics=("arbitrary",)),
    )(*operands)
    return tuple(outs)


# speedup vs baseline: 4.7254x; 3.5510x over previous
import functools

import jax
import jax.numpy as jnp
from jax import lax
from jax.experimental import pallas as pl
from jax.experimental.pallas import tpu as pltpu
from jax.experimental.pallas import tpu_sc as plsc

B, R, KT, C, H = 4096, 8, 2, 8, 128
K = KT + 1
BB = 128
NC, NS, L = 2, 16, 16
NW = NC * NS
BPW = B // NW


def _sc_gather_body(tgt_hbm, c0, c1, c2, c3, o0, o1, o2, o3,
                    tgt_v, idx_v, r0, r1, r2, r3, sem):
    wid = lax.axis_index("s") * NC + lax.axis_index("c")
    base = wid * BPW
    pltpu.sync_copy(tgt_hbm.at[pl.ds(base, BPW)], tgt_v)
    for i in range(BPW // L):
        tv = tgt_v[pl.ds(L * i, L)]
        bb = lax.iota(jnp.int32, L) + (base + L * i)
        idx_v[pl.ds(L * i, L)] = bb * C + tv
    copies = [pltpu.async_copy(c.at[idx_v], r, sem)
              for c, r in ((c0, r0), (c1, r1), (c2, r2), (c3, r3))]
    for cp in copies:
        cp.wait()
    for r, o in ((r0, o0), (r1, o1), (r2, o2), (r3, o3)):
        pltpu.sync_copy(r, o.at[pl.ds(base, BPW)])


def _sc_gather(targets, cands):
    row = jax.ShapeDtypeStruct((B, H), jnp.float32)
    fn = pl.kernel(
        _sc_gather_body,
        mesh=plsc.VectorSubcoreMesh(core_axis_name="c", subcore_axis_name="s"),
        out_type=[row] * 4,
        scratch_types=[
            pltpu.VMEM((BPW,), jnp.int32),
            pltpu.VMEM((BPW,), jnp.int32),
            pltpu.VMEM((BPW, H), jnp.float32),
            pltpu.VMEM((BPW, H), jnp.float32),
            pltpu.VMEM((BPW, H), jnp.float32),
            pltpu.VMEM((BPW, H), jnp.float32),
            pltpu.SemaphoreType.DMA,
        ],
    )
    return fn(targets, *[c.reshape(B * C, H) for c in cands])


def _fused_body(*refs):
    outs = refs[12:]
    ones_h = jnp.full((H, H), -1.0 / (K * H), dtype=jnp.float32)

    for a in range(4):
        o_ref, te_ref, ch_ref = refs[3 * a:3 * a + 3]
        o = o_ref[...]
        te = te_ref[...]
        chosen = ch_ref[...][:, None, :]
        tf = (te[:, 0:1, :], te[:, 1:2, :], chosen)

        acc = None
        for k in range(K):
            d = o[:, k] - tf[k]
            sq = d * d
            acc = sq if acc is None else acc + sq

        s = jnp.dot(acc.reshape(BB * R, H), ones_h,
                    preferred_element_type=jnp.float32).reshape(BB, R, H)
        m = jnp.max(s, axis=1, keepdims=True)
        e = jnp.exp(s - m)
        w = e / jnp.sum(e, axis=1, keepdims=True)

        outs[a][...] = jnp.stack(
            [jnp.sum(o[:, k] * w, axis=1) for k in range(K)], axis=0)


def kernel(outputs_position, tests_position, candidates_position,
           outputs_type, tests_type, candidates_type,
           outputs_size, tests_size, candidates_size,
           outputs_color, tests_color, candidates_color,
           targets):
    nb = B // BB
    chosen = _sc_gather(targets.astype(jnp.int32),
                        (candidates_position, candidates_type,
                         candidates_size, candidates_color))

    o_spec = pl.BlockSpec((BB, K, R, H), lambda i: (i, 0, 0, 0))
    t_spec = pl.BlockSpec((BB, KT, H), lambda i: (i, 0, 0))
    ch_spec = pl.BlockSpec((BB, H), lambda i: (i, 0))
    out_spec = pl.BlockSpec((K, BB, H), lambda i: (0, i, 0))

    in_specs = []
    operands = []
    for a, (o, te) in enumerate(
            ((outputs_position, tests_position),
             (outputs_type, tests_type),
             (outputs_size, tests_size),
             (outputs_color, tests_color))):
        in_specs += [o_spec, t_spec, ch_spec]
        operands += [jnp.transpose(o, (0, 2, 1, 3)), te, chosen[a]]

    out_shape = jax.ShapeDtypeStruct((K, B, H), jnp.float32)
    grid_spec = pl.GridSpec(
        grid=(nb,),
        in_specs=in_specs,
        out_specs=[out_spec] * 4,
    )
    outs = pl.pallas_call(
        _fused_body,
        grid_spec=grid_spec,
        out_shape=[out_shape] * 4,
        compiler_params=pltpu.CompilerParams(
            dimension_semantics=("arbitrary",)),
    )(*operands)
    return tuple(jnp.transpose(x, (1, 0, 2)) for x in outs)
